# baseline (speedup 1.0000x reference)
import jax
import jax.numpy as jnp
import numpy as np
from jax.experimental import pallas as pl
from jax.experimental.pallas import tpu as pltpu

NUM_CODES = 1024
DIM = 32
ROWS = 2048
A_ROWS = 2048
GRP = 8
PATCH_CAP = 1024
PATCH_GRP = 8
DELTA = np.float32(1e-5)
HI = jax.lax.Precision.HIGHEST


def _exact_dist(z, et):
    diff = z[:, :, None] - et[None, :, :]
    sq = diff * diff

    def _tile_sum(t):
        h = t[:, 0:4, :] + t[:, 4:8, :]
        h = h[:, 0:2, :] + h[:, 2:4, :]
        return h[:, 0, :] + h[:, 1, :]

    t0 = _tile_sum(sq[:, 0:8, :])
    t1 = _tile_sum(sq[:, 8:16, :])
    t2 = _tile_sum(sq[:, 16:24, :])
    t3 = _tile_sum(sq[:, 24:32, :])
    s = ((t0 + t1) + t2) + t3
    return jnp.sqrt(s)


def _exact_dist_t(zt, es):
    diff = zt - es
    sq = diff * diff

    def _tile_sum(t):
        h = t[0:4, :] + t[4:8, :]
        h = h[0:2, :] + h[2:4, :]
        return h[0, :] + h[1, :]

    t0 = _tile_sum(sq[0:8, :])
    t1 = _tile_sum(sq[8:16, :])
    t2 = _tile_sum(sq[16:24, :])
    t3 = _tile_sum(sq[24:32, :])
    return jnp.sqrt(((t0 + t1) + t2) + t3)


def _first_argmin(vals):
    m = jnp.min(vals, axis=1, keepdims=True)
    iota = jax.lax.broadcasted_iota(jnp.int32, vals.shape, 1)
    return m, iota, jnp.min(jnp.where(vals == m, iota, NUM_CODES), axis=1)


def _score_kernel(z_ref, zt_ref, et_ref, idx_ref, fb_ref, gsum_ref, out_ref,
                  acc_ref, e2_ref, w8_ref):
    @pl.when(pl.program_id(0) == 0)
    def _init():
        et = et_ref[...]
        e2_ref[...] = jnp.sum(et * et, axis=0, keepdims=True)
        acc_ref[...] = jnp.zeros((1, 1), jnp.float32)
        r8 = jax.lax.broadcasted_iota(jnp.int32, (A_ROWS, A_ROWS // GRP), 0)
        c8 = jax.lax.broadcasted_iota(jnp.int32, (A_ROWS, A_ROWS // GRP), 1)
        w8_ref[...] = (r8 // GRP == c8).astype(jnp.float32)

    z = z_ref[...]
    g = jnp.dot(z, et_ref[...], preferred_element_type=jnp.float32,
                precision=HI)
    s = e2_ref[...] - 2.0 * g

    m1 = jnp.min(s, axis=1)
    z2 = jnp.sum(z * z, axis=1)
    thr = m1 + DELTA * (z2 + 1.0)
    near = s < thr[:, None]
    iota = jax.lax.broadcasted_iota(jnp.int32, s.shape, 1)
    i1 = jnp.min(jnp.where(s == m1[:, None], iota, NUM_CODES), axis=1)
    isum = jnp.sum(jnp.where(near, iota, 0), axis=1)
    cnt = jnp.sum(near.astype(jnp.int32), axis=1)
    i2 = jnp.where(cnt == 2, isum - i1, i1)
    i2 = jnp.clip(i2, 0, NUM_CODES - 1)
    fbf = (cnt >= 3).astype(jnp.float32)

    zt = zt_ref[...]
    iota_c = jax.lax.broadcasted_iota(jnp.int32, (NUM_CODES, A_ROWS), 0)
    oh1 = (iota_c == i1[None, :]).astype(jnp.float32)
    oh2 = (iota_c == i2[None, :]).astype(jnp.float32)
    es1 = jnp.dot(et_ref[...], oh1, preferred_element_type=jnp.float32,
                  precision=HI)
    es2 = jnp.dot(et_ref[...], oh2, preferred_element_type=jnp.float32,
                  precision=HI)
    d1 = _exact_dist_t(zt, es1)
    d2 = _exact_dist_t(zt, es2)
    take2 = (d2 < d1) | ((d2 == d1) & (i2 < i1))
    idx = jnp.where(take2, i2, i1)

    idx_ref[0, 0, :] = idx
    fb_ref[0, 0, :] = fbf.astype(jnp.int32)
    gs = jnp.dot(fbf.reshape(1, A_ROWS), w8_ref[...],
                 preferred_element_type=jnp.float32, precision=HI)
    gsum_ref[0, 0, :] = gs[0].astype(jnp.int32)
    out_ref[...] = z
    acc_ref[...] += jnp.sum(z2 + m1).reshape(1, 1)


def _patch_kernel(gsum_sm, flags_sm, z_ref, et_ref, prov_ref, out_ref,
                  ids_sm, cnt_sm):
    out_ref[...] = prov_ref[...]

    def _inner(r, cc):
        j = cc[0] * GRP + r
        f = flags_sm[j]
        c = cc[1]

        @pl.when((f != 0) & (c < PATCH_CAP))
        def _():
            ids_sm[c] = j

        return (cc[0], c + jnp.where(f != 0, 1, 0))

    def _outer(gi, c):
        trip = jnp.where(gsum_sm[gi] != 0, GRP, 0)
        _, c2 = jax.lax.fori_loop(0, trip, _inner, (gi, c))
        return c2

    cnt = jax.lax.fori_loop(0, ROWS // GRP, _outer, 0)
    cnt = jnp.minimum(cnt, PATCH_CAP)
    cnt_sm[0] = cnt
    ngrp = (cnt + PATCH_GRP - 1) // PATCH_GRP

    def _grp(gi, carry):
        base = gi * PATCH_GRP
        rows = [
            jnp.where(base + r < cnt, ids_sm[base + r], 0)
            for r in range(PATCH_GRP)
        ]
        zc = jnp.concatenate(
            [z_ref[pl.ds(rows[r], 1), :] for r in range(PATCH_GRP)], axis=0
        )
        dist = _exact_dist(zc, et_ref[...])
        _, _, idx = _first_argmin(dist)
        idx2 = idx.reshape(PATCH_GRP, 1)
        for r in range(PATCH_GRP):
            out_ref[pl.ds(rows[r], 1), :] = idx2[r : r + 1, :]
        return carry

    jax.lax.fori_loop(0, ngrp, _grp, 0)


@jax.jit
def kernel(z_e, embeddings):
    z_t = jnp.transpose(z_e, (0, 3, 1, 2))
    z_flat = z_t.reshape(ROWS, DIM)
    z_flat_t = z_flat.T
    et = embeddings.T

    grid = ROWS // A_ROWS
    idx3, fb3, gsum3, out, acc = pl.pallas_call(
        _score_kernel,
        grid=(grid,),
        in_specs=[
            pl.BlockSpec((A_ROWS, DIM), lambda i: (i, 0)),
            pl.BlockSpec((DIM, A_ROWS), lambda i: (0, i)),
            pl.BlockSpec((DIM, NUM_CODES), lambda i: (0, 0)),
        ],
        out_specs=[
            pl.BlockSpec((1, 1, A_ROWS), lambda i: (i, 0, 0)),
            pl.BlockSpec((1, 1, A_ROWS), lambda i: (i, 0, 0)),
            pl.BlockSpec((1, 1, A_ROWS // GRP), lambda i: (i, 0, 0)),
            pl.BlockSpec((A_ROWS, DIM), lambda i: (i, 0)),
            pl.BlockSpec((1, 1), lambda i: (0, 0)),
        ],
        out_shape=[
            jax.ShapeDtypeStruct((grid, 1, A_ROWS), jnp.int32),
            jax.ShapeDtypeStruct((grid, 1, A_ROWS), jnp.int32),
            jax.ShapeDtypeStruct((grid, 1, A_ROWS // GRP), jnp.int32),
            jax.ShapeDtypeStruct((ROWS, DIM), jnp.float32),
            jax.ShapeDtypeStruct((1, 1), jnp.float32),
        ],
        scratch_shapes=[
            pltpu.VMEM((1, NUM_CODES), jnp.float32),
            pltpu.VMEM((A_ROWS, A_ROWS // GRP), jnp.float32),
        ],
    )(z_flat, z_flat_t, et)

    gsums = gsum3.reshape(ROWS // GRP)
    flags = fb3.reshape(ROWS)
    prov = idx3.reshape(ROWS, 1)

    idx_fixed = pl.pallas_call(
        _patch_kernel,
        grid_spec=pltpu.PrefetchScalarGridSpec(
            num_scalar_prefetch=2,
            grid=(1,),
            in_specs=[
                pl.BlockSpec((ROWS, DIM), lambda i, g, f: (0, 0)),
                pl.BlockSpec((DIM, NUM_CODES), lambda i, g, f: (0, 0)),
                pl.BlockSpec((ROWS, 1), lambda i, g, f: (0, 0)),
            ],
            out_specs=pl.BlockSpec((ROWS, 1), lambda i, g, f: (0, 0)),
            scratch_shapes=[
                pltpu.SMEM((PATCH_CAP,), jnp.int32),
                pltpu.SMEM((1,), jnp.int32),
            ],
        ),
        out_shape=jax.ShapeDtypeStruct((ROWS, 1), jnp.int32),
    )(gsums, flags, z_flat, et, prov)

    indices = idx_fixed.reshape(ROWS)
    zqwg = jnp.transpose(out.reshape(z_t.shape), (0, 2, 3, 1))
    loss = acc[0, 0] * np.float32(0.25 / 65536.0)
    return zqwg, indices, loss


# speedup vs baseline: 1.0297x; 1.0297x over previous
import jax
import jax.numpy as jnp
import numpy as np
from jax.experimental import pallas as pl
from jax.experimental.pallas import tpu as pltpu

NUM_CODES = 1024
DIM = 32
ROWS = 2048
A_ROWS = 1024
GRP = 8
PATCH_CAP = 1024
PATCH_GRP = 8
DELTA = np.float32(1e-5)
HI = jax.lax.Precision.HIGHEST


def _exact_dist(z, et):
    diff = z[:, :, None] - et[None, :, :]
    sq = diff * diff

    def _tile_sum(t):
        h = t[:, 0:4, :] + t[:, 4:8, :]
        h = h[:, 0:2, :] + h[:, 2:4, :]
        return h[:, 0, :] + h[:, 1, :]

    t0 = _tile_sum(sq[:, 0:8, :])
    t1 = _tile_sum(sq[:, 8:16, :])
    t2 = _tile_sum(sq[:, 16:24, :])
    t3 = _tile_sum(sq[:, 24:32, :])
    s = ((t0 + t1) + t2) + t3
    return jnp.sqrt(s)


def _exact_dist_t(zt, es):
    diff = zt - es
    sq = diff * diff

    def _tile_sum(t):
        h = t[0:4, :] + t[4:8, :]
        h = h[0:2, :] + h[2:4, :]
        return h[0, :] + h[1, :]

    t0 = _tile_sum(sq[0:8, :])
    t1 = _tile_sum(sq[8:16, :])
    t2 = _tile_sum(sq[16:24, :])
    t3 = _tile_sum(sq[24:32, :])
    return jnp.sqrt(((t0 + t1) + t2) + t3)


def _first_argmin(vals):
    m = jnp.min(vals, axis=1, keepdims=True)
    iota = jax.lax.broadcasted_iota(jnp.int32, vals.shape, 1)
    return m, iota, jnp.min(jnp.where(vals == m, iota, NUM_CODES), axis=1)


def _score_kernel(z_ref, zt_ref, et_ref, idx_ref, fb_ref, gsum_ref, out_ref,
                  acc_ref, e2_ref, w8_ref):
    @pl.when(pl.program_id(0) == 0)
    def _init():
        et = et_ref[...]
        e2_ref[...] = jnp.sum(et * et, axis=0, keepdims=True)
        acc_ref[...] = jnp.zeros((1, 1), jnp.float32)
        r8 = jax.lax.broadcasted_iota(jnp.int32, (A_ROWS, A_ROWS // GRP), 0)
        c8 = jax.lax.broadcasted_iota(jnp.int32, (A_ROWS, A_ROWS // GRP), 1)
        w8_ref[...] = (r8 // GRP == c8).astype(jnp.float32)

    z = z_ref[...]
    g = jnp.dot(z, et_ref[...], preferred_element_type=jnp.float32,
                precision=HI)
    s = e2_ref[...] - 2.0 * g

    m1 = jnp.min(s, axis=1)
    z2 = jnp.sum(z * z, axis=1)
    thr = m1 + DELTA * (z2 + 1.0)
    near = s < thr[:, None]
    iota = jax.lax.broadcasted_iota(jnp.int32, s.shape, 1)
    i1 = jnp.min(jnp.where(s == m1[:, None], iota, NUM_CODES), axis=1)
    isum = jnp.sum(jnp.where(near, iota, 0), axis=1)
    cnt = jnp.sum(near.astype(jnp.int32), axis=1)
    i2 = jnp.where(cnt == 2, isum - i1, i1)
    i2 = jnp.clip(i2, 0, NUM_CODES - 1)
    fbf = (cnt >= 3).astype(jnp.float32)

    zt = zt_ref[...]
    iota_c = jax.lax.broadcasted_iota(jnp.int32, (NUM_CODES, A_ROWS), 0)
    oh1 = (iota_c == i1[None, :]).astype(jnp.float32)
    oh2 = (iota_c == i2[None, :]).astype(jnp.float32)
    es1 = jnp.dot(et_ref[...], oh1, preferred_element_type=jnp.float32,
                  precision=HI)
    es2 = jnp.dot(et_ref[...], oh2, preferred_element_type=jnp.float32,
                  precision=HI)
    d1 = _exact_dist_t(zt, es1)
    d2 = _exact_dist_t(zt, es2)
    take2 = (d2 < d1) | ((d2 == d1) & (i2 < i1))
    idx = jnp.where(take2, i2, i1)

    idx_ref[0, 0, :] = idx
    fb_ref[0, 0, :] = fbf.astype(jnp.int32)
    gs = jnp.dot(fbf.reshape(1, A_ROWS), w8_ref[...],
                 preferred_element_type=jnp.float32, precision=HI)
    gsum_ref[0, 0, :] = gs[0].astype(jnp.int32)
    out_ref[...] = z
    acc_ref[...] += jnp.sum(z2 + m1).reshape(1, 1)


def _patch_kernel(gsum_sm, flags_sm, z_ref, et_ref, prov_ref, out_ref,
                  ids_sm, cnt_sm):
    out_ref[...] = prov_ref[...]

    def _inner(r, cc):
        j = cc[0] * GRP + r
        f = flags_sm[j]
        c = cc[1]

        @pl.when((f != 0) & (c < PATCH_CAP))
        def _():
            ids_sm[c] = j

        return (cc[0], c + jnp.where(f != 0, 1, 0))

    def _outer(gi, c):
        trip = jnp.where(gsum_sm[gi] != 0, GRP, 0)
        _, c2 = jax.lax.fori_loop(0, trip, _inner, (gi, c))
        return c2

    cnt = jax.lax.fori_loop(0, ROWS // GRP, _outer, 0)
    cnt = jnp.minimum(cnt, PATCH_CAP)
    cnt_sm[0] = cnt
    ngrp = (cnt + PATCH_GRP - 1) // PATCH_GRP

    def _grp(gi, carry):
        base = gi * PATCH_GRP
        rows = [
            jnp.where(base + r < cnt, ids_sm[base + r], 0)
            for r in range(PATCH_GRP)
        ]
        zc = jnp.concatenate(
            [z_ref[pl.ds(rows[r], 1), :] for r in range(PATCH_GRP)], axis=0
        )
        dist = _exact_dist(zc, et_ref[...])
        _, _, idx = _first_argmin(dist)
        idx2 = idx.reshape(PATCH_GRP, 1)
        for r in range(PATCH_GRP):
            out_ref[pl.ds(rows[r], 1), :] = idx2[r : r + 1, :]
        return carry

    jax.lax.fori_loop(0, ngrp, _grp, 0)


@jax.jit
def kernel(z_e, embeddings):
    z_t = jnp.transpose(z_e, (0, 3, 1, 2))
    z_flat = z_t.reshape(ROWS, DIM)
    z_flat_t = z_flat.T
    et = embeddings.T

    grid = ROWS // A_ROWS
    idx3, fb3, gsum3, out, acc = pl.pallas_call(
        _score_kernel,
        grid=(grid,),
        in_specs=[
            pl.BlockSpec((A_ROWS, DIM), lambda i: (i, 0)),
            pl.BlockSpec((DIM, A_ROWS), lambda i: (0, i)),
            pl.BlockSpec((DIM, NUM_CODES), lambda i: (0, 0)),
        ],
        out_specs=[
            pl.BlockSpec((1, 1, A_ROWS), lambda i: (i, 0, 0)),
            pl.BlockSpec((1, 1, A_ROWS), lambda i: (i, 0, 0)),
            pl.BlockSpec((1, 1, A_ROWS // GRP), lambda i: (i, 0, 0)),
            pl.BlockSpec((A_ROWS, DIM), lambda i: (i, 0)),
            pl.BlockSpec((1, 1), lambda i: (0, 0)),
        ],
        out_shape=[
            jax.ShapeDtypeStruct((grid, 1, A_ROWS), jnp.int32),
            jax.ShapeDtypeStruct((grid, 1, A_ROWS), jnp.int32),
            jax.ShapeDtypeStruct((grid, 1, A_ROWS // GRP), jnp.int32),
            jax.ShapeDtypeStruct((ROWS, DIM), jnp.float32),
            jax.ShapeDtypeStruct((1, 1), jnp.float32),
        ],
        scratch_shapes=[
            pltpu.VMEM((1, NUM_CODES), jnp.float32),
            pltpu.VMEM((A_ROWS, A_ROWS // GRP), jnp.float32),
        ],
    )(z_flat, z_flat_t, et)

    gsums = gsum3.reshape(ROWS // GRP)
    flags = fb3.reshape(ROWS)
    prov = idx3.reshape(ROWS, 1)

    idx_fixed = pl.pallas_call(
        _patch_kernel,
        grid_spec=pltpu.PrefetchScalarGridSpec(
            num_scalar_prefetch=2,
            grid=(1,),
            in_specs=[
                pl.BlockSpec((ROWS, DIM), lambda i, g, f: (0, 0)),
                pl.BlockSpec((DIM, NUM_CODES), lambda i, g, f: (0, 0)),
                pl.BlockSpec((ROWS, 1), lambda i, g, f: (0, 0)),
            ],
            out_specs=pl.BlockSpec((ROWS, 1), lambda i, g, f: (0, 0)),
            scratch_shapes=[
                pltpu.SMEM((PATCH_CAP,), jnp.int32),
                pltpu.SMEM((1,), jnp.int32),
            ],
        ),
        out_shape=jax.ShapeDtypeStruct((ROWS, 1), jnp.int32),
    )(gsums, flags, z_flat, et, prov)

    indices = idx_fixed.reshape(ROWS)
    zqwg = jnp.transpose(out.reshape(z_t.shape), (0, 2, 3, 1))
    loss = acc[0, 0] * np.float32(0.25 / 65536.0)
    return zqwg, indices, loss
